# Initial kernel scaffold; baseline (speedup 1.0000x reference)
import jax
import jax.numpy as jnp
from jax.experimental import pallas as pl


def kernel(h, e, edge_index, action, Wn, We, A, B, C, U, V, W1, b1, W2, b2):
    raise NotImplementedError("write your pallas kernel here")


# speedup vs baseline: 3.0325x; 3.0325x over previous
import functools

import jax
import jax.numpy as jnp
from jax import lax
from jax.experimental import pallas as pl
from jax.experimental.pallas import tpu as pltpu
from jax.experimental.pallas import tpu_sc as plsc

_N = 10000
_NP = 10240
_E = 320000
_H = 128
_DE = 16
_AD = 8

_SCI = plsc.get_sparse_core_info()
_NC = _SCI.num_cores
_NS = _SCI.num_subcores
_NW = _NC * _NS

_BN = 1000
_BE = 2560
_CH = 80

_f32 = jnp.float32


def _proj_body(h_ref, Wn_ref, A_ref, B_ref, V_ref, U_ref,
               h0_ref, td_ref, ts_ref, hu_ref):
    h0 = jnp.dot(h_ref[...], Wn_ref[...], preferred_element_type=_f32)
    h0_ref[...] = h0
    td_ref[...] = jnp.dot(h0, A_ref[...], preferred_element_type=_f32)
    ts_ref[:, :_H] = jnp.dot(h0, B_ref[...], preferred_element_type=_f32)
    ts_ref[:, _H:] = jnp.dot(h0, V_ref[...], preferred_element_type=_f32)
    hu_ref[...] = jnp.dot(h0, U_ref[...], preferred_element_type=_f32)


def _tc_proj(h, Wn, A0, B0, V0, U0):
    n_blk = pl.BlockSpec((_BN, _H), lambda i: (i, 0))
    w_blk = pl.BlockSpec((_H, _H), lambda i: (0, 0))
    return pl.pallas_call(
        _proj_body,
        grid=(_N // _BN,),
        in_specs=[n_blk, w_blk, w_blk, w_blk, w_blk, w_blk],
        out_specs=[n_blk, n_blk, pl.BlockSpec((_BN, 2 * _H), lambda i: (i, 0)),
                   n_blk],
        out_shape=[
            jax.ShapeDtypeStruct((_N, _H), _f32),
            jax.ShapeDtypeStruct((_N, _H), _f32),
            jax.ShapeDtypeStruct((_N, 2 * _H), _f32),
            jax.ShapeDtypeStruct((_N, _H), _f32),
        ],
    )(h, Wn, A0, B0, V0, U0)


def _upd_proj_body(h_ref, num_ref, den_ref, hu_ref,
                   A_ref, B_ref, V_ref, U_ref,
                   h1_ref, td_ref, ts_ref, hu1_ref):
    agg = num_ref[...] / (den_ref[...] + 1e-6)
    h1 = h_ref[...] + jnp.maximum(hu_ref[...] + agg, 0.0)
    h1_ref[...] = h1
    td_ref[...] = jnp.dot(h1, A_ref[...], preferred_element_type=_f32)
    ts_ref[:, :_H] = jnp.dot(h1, B_ref[...], preferred_element_type=_f32)
    ts_ref[:, _H:] = jnp.dot(h1, V_ref[...], preferred_element_type=_f32)
    hu1_ref[...] = jnp.dot(h1, U_ref[...], preferred_element_type=_f32)


def _tc_upd_proj(h0, num0, den0, hu0, A1, B1, V1, U1):
    n_blk = pl.BlockSpec((_BN, _H), lambda i: (i, 0))
    w_blk = pl.BlockSpec((_H, _H), lambda i: (0, 0))
    return pl.pallas_call(
        _upd_proj_body,
        grid=(_N // _BN,),
        in_specs=[n_blk, n_blk, n_blk, n_blk, w_blk, w_blk, w_blk, w_blk],
        out_specs=[n_blk, n_blk, pl.BlockSpec((_BN, 2 * _H), lambda i: (i, 0)),
                   n_blk],
        out_shape=[
            jax.ShapeDtypeStruct((_N, _H), _f32),
            jax.ShapeDtypeStruct((_N, _H), _f32),
            jax.ShapeDtypeStruct((_N, 2 * _H), _f32),
            jax.ShapeDtypeStruct((_N, _H), _f32),
        ],
    )(h0, num0, den0, hu0, A1, B1, V1, U1)


def _mid0_body(er_ref, gA_ref, gBV_ref, We_ref, C0_ref, C1_ref,
               msg_ref, sig_ref, eC1_ref):
    P0 = jnp.dot(We_ref[...], C0_ref[...], preferred_element_type=_f32)
    P1 = jnp.dot(We_ref[...], C1_ref[...], preferred_element_type=_f32)
    er = er_ref[...]
    ehat = (jnp.dot(er, P0, preferred_element_type=_f32)
            + gA_ref[...] + gBV_ref[:, :_H])
    sig = jax.nn.sigmoid(ehat)
    sig_ref[...] = sig
    msg_ref[...] = sig * gBV_ref[:, _H:]
    r = jnp.maximum(ehat, 0.0)
    eC1_ref[...] = (jnp.dot(er, P1, preferred_element_type=_f32)
                    + jnp.dot(r, C1_ref[...], preferred_element_type=_f32))


def _tc_mid0(e_raw, gA, gBV, We, C0, C1):
    e_blk = pl.BlockSpec((_BE, _H), lambda i: (i, 0))
    return pl.pallas_call(
        _mid0_body,
        grid=(_E // _BE,),
        in_specs=[
            pl.BlockSpec((_BE, _DE), lambda i: (i, 0)),
            e_blk,
            pl.BlockSpec((_BE, 2 * _H), lambda i: (i, 0)),
            pl.BlockSpec((_DE, _H), lambda i: (0, 0)),
            pl.BlockSpec((_H, _H), lambda i: (0, 0)),
            pl.BlockSpec((_H, _H), lambda i: (0, 0)),
        ],
        out_specs=[e_blk, e_blk, e_blk],
        out_shape=[
            jax.ShapeDtypeStruct((_E, _H), _f32),
            jax.ShapeDtypeStruct((_E, _H), _f32),
            jax.ShapeDtypeStruct((_E, _H), _f32),
        ],
    )(e_raw, gA, gBV, We, C0, C1)


def _mid1_body(eC1_ref, gA_ref, gBV_ref, msg_ref, sig_ref):
    ehat = eC1_ref[...] + gA_ref[...] + gBV_ref[:, :_H]
    sig = jax.nn.sigmoid(ehat)
    sig_ref[...] = sig
    msg_ref[...] = sig * gBV_ref[:, _H:]


def _tc_mid1(eC1, gA, gBV):
    e_blk = pl.BlockSpec((_BE, _H), lambda i: (i, 0))
    return pl.pallas_call(
        _mid1_body,
        grid=(_E // _BE,),
        in_specs=[e_blk, e_blk, pl.BlockSpec((_BE, 2 * _H), lambda i: (i, 0))],
        out_specs=[e_blk, e_blk],
        out_shape=[
            jax.ShapeDtypeStruct((_E, _H), _f32),
            jax.ShapeDtypeStruct((_E, _H), _f32),
        ],
    )(eC1, gA, gBV)


def _head_body(h_ref, num_ref, den_ref, hu_ref, act_ref,
               W1h_ref, W1a_ref, b1_ref, W2_ref, b2_ref, out_ref):
    i = pl.program_id(0)
    agg = num_ref[...] / (den_ref[...] + 1e-6)
    h2 = h_ref[...] + jnp.maximum(hu_ref[...] + agg, 0.0)
    z = jnp.maximum(
        jnp.dot(h2, W1h_ref[...], preferred_element_type=_f32)
        + jnp.dot(act_ref[...], W1a_ref[...], preferred_element_type=_f32)
        + b1_ref[...], 0.0)
    y = jnp.dot(z, W2_ref[...], preferred_element_type=_f32) + b2_ref[...]

    @pl.when(i == 0)
    def _():
        out_ref[...] = jnp.zeros_like(out_ref)

    out_ref[...] += jnp.reshape(jnp.sum(y) / _N, (1, 1))


def _tc_head(h1, num1, den1, hu1, action, W1h, W1a, b1, W2, b2):
    n_blk = pl.BlockSpec((_BN, _H), lambda i: (i, 0))
    return pl.pallas_call(
        _head_body,
        grid=(_N // _BN,),
        in_specs=[
            n_blk, n_blk, n_blk, n_blk,
            pl.BlockSpec((_BN, _AD), lambda i: (i, 0)),
            pl.BlockSpec((_H, _H), lambda i: (0, 0)),
            pl.BlockSpec((_AD, _H), lambda i: (0, 0)),
            pl.BlockSpec((1, _H), lambda i: (0, 0)),
            pl.BlockSpec((_H, 1), lambda i: (0, 0)),
            pl.BlockSpec((1, 1), lambda i: (0, 0)),
        ],
        out_specs=pl.BlockSpec((1, 1), lambda i: (0, 0)),
        out_shape=jax.ShapeDtypeStruct((1, 1), _f32),
    )(h1, num1, den1, hu1, action, W1h, W1a, b1, W2, b2)


_MESH = plsc.VectorSubcoreMesh(core_axis_name="c", subcore_axis_name="s")


def _sc_gather_body(td_hbm, ts_hbm, dst_hbm, src_hbm, gA_hbm, gBV_hbm,
                    idx_d, idx_s, bufA, bufBV, sem0, sem1, sem2, sem3):
    c = lax.axis_index("c")
    s = lax.axis_index("s")
    wid = s * _NC + c
    ew = _E // _NW
    nch = ew // _CH

    def chunk(i, carry):
        base = wid * ew + i * _CH
        cp0 = pltpu.async_copy(dst_hbm.at[pl.ds(base, _CH)], idx_d, sem0)
        cp1 = pltpu.async_copy(src_hbm.at[pl.ds(base, _CH)], idx_s, sem1)
        cp0.wait()
        cp1.wait()
        g0 = pltpu.async_copy(td_hbm.at[idx_d], bufA, sem2)
        g1 = pltpu.async_copy(ts_hbm.at[idx_s], bufBV, sem3)
        g0.wait()
        g1.wait()
        w0 = pltpu.async_copy(bufA, gA_hbm.at[pl.ds(base, _CH)], sem0)
        w1 = pltpu.async_copy(bufBV, gBV_hbm.at[pl.ds(base, _CH)], sem1)
        w0.wait()
        w1.wait()
        return carry

    lax.fori_loop(0, nch, chunk, 0)


@functools.partial(
    pl.kernel,
    out_type=[
        jax.ShapeDtypeStruct((_E, _H), _f32),
        jax.ShapeDtypeStruct((_E, 2 * _H), _f32),
    ],
    mesh=_MESH,
    scratch_types=[
        pltpu.VMEM((_CH,), jnp.int32),
        pltpu.VMEM((_CH,), jnp.int32),
        pltpu.VMEM((_CH, _H), _f32),
        pltpu.VMEM((_CH, 2 * _H), _f32),
        pltpu.SemaphoreType.DMA,
        pltpu.SemaphoreType.DMA,
        pltpu.SemaphoreType.DMA,
        pltpu.SemaphoreType.DMA,
    ],
)
def _sc_gather(td_hbm, ts_hbm, dst_hbm, src_hbm, gA_hbm, gBV_hbm,
               idx_d, idx_s, bufA, bufBV, sem0, sem1, sem2, sem3):
    _sc_gather_body(td_hbm, ts_hbm, dst_hbm, src_hbm, gA_hbm, gBV_hbm,
                    idx_d, idx_s, bufA, bufBV, sem0, sem1, sem2, sem3)


def _sc_scatter_stream(data_hbm, dst_hbm, acc, dbuf, idxbuf, s):
    ew = _E // _NS
    nch = ew // _CH

    def chunk(i, carry):
        base = s * ew + i * _CH
        pltpu.sync_copy(data_hbm.at[pl.ds(base, _CH)], dbuf)
        pltpu.sync_copy(dst_hbm.at[pl.ds(base, _CH)], idxbuf)
        pltpu.sync_copy(dbuf, acc.at[idxbuf], add=True)
        return carry

    lax.fori_loop(0, nch, chunk, 0)


@functools.partial(
    pl.kernel,
    out_type=[
        jax.ShapeDtypeStruct((_NP, _H), _f32),
        jax.ShapeDtypeStruct((_NP, _H), _f32),
    ],
    mesh=_MESH,
    scratch_types=[
        pltpu.VMEM((_CH, _H), _f32),
        pltpu.VMEM((_CH,), jnp.int32),
        pltpu.VMEM_SHARED((_NP, _H), _f32),
    ],
)
def _sc_scatter(msg_hbm, sig_hbm, dst_hbm, zeros_hbm, num_hbm, den_hbm,
                dbuf, idxbuf, acc):
    c = lax.axis_index("c")
    s = lax.axis_index("s")
    rows = _NP // _NS
    pltpu.sync_copy(zeros_hbm.at[pl.ds(s * rows, rows)],
                    acc.at[pl.ds(s * rows, rows)])
    plsc.subcore_barrier()

    @pl.when(c == 0)
    def _():
        _sc_scatter_stream(msg_hbm, dst_hbm, acc, dbuf, idxbuf, s)

    @pl.when(c == 1)
    def _():
        _sc_scatter_stream(sig_hbm, dst_hbm, acc, dbuf, idxbuf, s)

    plsc.subcore_barrier()

    @pl.when(c == 0)
    def _():
        pltpu.sync_copy(acc.at[pl.ds(s * rows, rows)],
                        num_hbm.at[pl.ds(s * rows, rows)])

    @pl.when(c == 1)
    def _():
        pltpu.sync_copy(acc.at[pl.ds(s * rows, rows)],
                        den_hbm.at[pl.ds(s * rows, rows)])


def kernel(h, e, edge_index, action, Wn, We, A, B, C, U, V, W1, b1, W2, b2):
    src = edge_index[0]
    dst = edge_index[1]
    zeros_n = jnp.zeros((_NP, _H), _f32)

    h0, td0, ts0, hu0 = _tc_proj(h, Wn, A[0], B[0], V[0], U[0])
    gA0, gBV0 = _sc_gather(td0, ts0, dst, src)
    msg0, sig0, eC1 = _tc_mid0(e, gA0, gBV0, We, C[0], C[1])
    num0, den0 = _sc_scatter(msg0, sig0, dst, zeros_n)

    h1, td1, ts1, hu1 = _tc_upd_proj(h0, num0, den0, hu0,
                                     A[1], B[1], V[1], U[1])
    gA1, gBV1 = _sc_gather(td1, ts1, dst, src)
    msg1, sig1 = _tc_mid1(eC1, gA1, gBV1)
    num1, den1 = _sc_scatter(msg1, sig1, dst, zeros_n)

    return _tc_head(h1, num1, den1, hu1, action,
                    W1[:_H], W1[_H:], b1.reshape(1, _H),
                    W2, b2.reshape(1, 1))


# speedup vs baseline: 3.8022x; 1.2538x over previous
import functools

import jax
import jax.numpy as jnp
from jax import lax
from jax.experimental import pallas as pl
from jax.experimental.pallas import tpu as pltpu
from jax.experimental.pallas import tpu_sc as plsc

_N = 10000
_NP = 10240
_E = 320000
_H = 128
_DE = 16
_AD = 8

_SCI = plsc.get_sparse_core_info()
_NC = _SCI.num_cores
_NS = _SCI.num_subcores
_NW = _NC * _NS

_BN = 1000
_BE = 2560
_CH = 80

_f32 = jnp.float32


def _proj_body(h_ref, Wn_ref, A_ref, B_ref, V_ref, U_ref,
               h0_ref, td_ref, ts_ref, hu_ref):
    h0 = jnp.dot(h_ref[...], Wn_ref[...], preferred_element_type=_f32)
    h0_ref[...] = h0
    td_ref[...] = jnp.dot(h0, A_ref[...], preferred_element_type=_f32)
    ts_ref[:, :_H] = jnp.dot(h0, B_ref[...], preferred_element_type=_f32)
    ts_ref[:, _H:] = jnp.dot(h0, V_ref[...], preferred_element_type=_f32)
    hu_ref[...] = jnp.dot(h0, U_ref[...], preferred_element_type=_f32)


def _tc_proj(h, Wn, A0, B0, V0, U0):
    n_blk = pl.BlockSpec((_BN, _H), lambda i: (i, 0))
    w_blk = pl.BlockSpec((_H, _H), lambda i: (0, 0))
    return pl.pallas_call(
        _proj_body,
        grid=(_N // _BN,),
        in_specs=[n_blk, w_blk, w_blk, w_blk, w_blk, w_blk],
        out_specs=[n_blk, n_blk, pl.BlockSpec((_BN, 2 * _H), lambda i: (i, 0)),
                   n_blk],
        out_shape=[
            jax.ShapeDtypeStruct((_N, _H), _f32),
            jax.ShapeDtypeStruct((_N, _H), _f32),
            jax.ShapeDtypeStruct((_N, 2 * _H), _f32),
            jax.ShapeDtypeStruct((_N, _H), _f32),
        ],
    )(h, Wn, A0, B0, V0, U0)


def _upd_proj_body(h_ref, num_ref, den_ref, hu_ref,
                   A_ref, B_ref, V_ref, U_ref,
                   h1_ref, td_ref, ts_ref, hu1_ref):
    agg = num_ref[...] / (den_ref[...] + 1e-6)
    h1 = h_ref[...] + jnp.maximum(hu_ref[...] + agg, 0.0)
    h1_ref[...] = h1
    td_ref[...] = jnp.dot(h1, A_ref[...], preferred_element_type=_f32)
    ts_ref[:, :_H] = jnp.dot(h1, B_ref[...], preferred_element_type=_f32)
    ts_ref[:, _H:] = jnp.dot(h1, V_ref[...], preferred_element_type=_f32)
    hu1_ref[...] = jnp.dot(h1, U_ref[...], preferred_element_type=_f32)


def _tc_upd_proj(h0, num0, den0, hu0, A1, B1, V1, U1):
    n_blk = pl.BlockSpec((_BN, _H), lambda i: (i, 0))
    w_blk = pl.BlockSpec((_H, _H), lambda i: (0, 0))
    return pl.pallas_call(
        _upd_proj_body,
        grid=(_N // _BN,),
        in_specs=[n_blk, n_blk, n_blk, n_blk, w_blk, w_blk, w_blk, w_blk],
        out_specs=[n_blk, n_blk, pl.BlockSpec((_BN, 2 * _H), lambda i: (i, 0)),
                   n_blk],
        out_shape=[
            jax.ShapeDtypeStruct((_N, _H), _f32),
            jax.ShapeDtypeStruct((_N, _H), _f32),
            jax.ShapeDtypeStruct((_N, 2 * _H), _f32),
            jax.ShapeDtypeStruct((_N, _H), _f32),
        ],
    )(h0, num0, den0, hu0, A1, B1, V1, U1)


def _mid0_body(er_ref, gA_ref, gBV_ref, We_ref, C0_ref, C1_ref,
               msg_ref, sig_ref, eC1_ref):
    P0 = jnp.dot(We_ref[...], C0_ref[...], preferred_element_type=_f32)
    P1 = jnp.dot(We_ref[...], C1_ref[...], preferred_element_type=_f32)
    er = er_ref[...]
    ehat = (jnp.dot(er, P0, preferred_element_type=_f32)
            + gA_ref[...] + gBV_ref[:, :_H])
    sig = jax.nn.sigmoid(ehat)
    sig_ref[...] = sig
    msg_ref[...] = sig * gBV_ref[:, _H:]
    r = jnp.maximum(ehat, 0.0)
    eC1_ref[...] = (jnp.dot(er, P1, preferred_element_type=_f32)
                    + jnp.dot(r, C1_ref[...], preferred_element_type=_f32))


def _tc_mid0(e_raw, gA, gBV, We, C0, C1):
    e_blk = pl.BlockSpec((_BE, _H), lambda i: (i, 0))
    return pl.pallas_call(
        _mid0_body,
        grid=(_E // _BE,),
        in_specs=[
            pl.BlockSpec((_BE, _DE), lambda i: (i, 0)),
            e_blk,
            pl.BlockSpec((_BE, 2 * _H), lambda i: (i, 0)),
            pl.BlockSpec((_DE, _H), lambda i: (0, 0)),
            pl.BlockSpec((_H, _H), lambda i: (0, 0)),
            pl.BlockSpec((_H, _H), lambda i: (0, 0)),
        ],
        out_specs=[e_blk, e_blk, e_blk],
        out_shape=[
            jax.ShapeDtypeStruct((_E, _H), _f32),
            jax.ShapeDtypeStruct((_E, _H), _f32),
            jax.ShapeDtypeStruct((_E, _H), _f32),
        ],
    )(e_raw, gA, gBV, We, C0, C1)


def _mid1_body(eC1_ref, gA_ref, gBV_ref, msg_ref, sig_ref):
    ehat = eC1_ref[...] + gA_ref[...] + gBV_ref[:, :_H]
    sig = jax.nn.sigmoid(ehat)
    sig_ref[...] = sig
    msg_ref[...] = sig * gBV_ref[:, _H:]


def _tc_mid1(eC1, gA, gBV):
    e_blk = pl.BlockSpec((_BE, _H), lambda i: (i, 0))
    return pl.pallas_call(
        _mid1_body,
        grid=(_E // _BE,),
        in_specs=[e_blk, e_blk, pl.BlockSpec((_BE, 2 * _H), lambda i: (i, 0))],
        out_specs=[e_blk, e_blk],
        out_shape=[
            jax.ShapeDtypeStruct((_E, _H), _f32),
            jax.ShapeDtypeStruct((_E, _H), _f32),
        ],
    )(eC1, gA, gBV)


def _head_body(h_ref, num_ref, den_ref, hu_ref, act_ref,
               W1h_ref, W1a_ref, b1_ref, W2_ref, b2_ref, out_ref):
    i = pl.program_id(0)
    agg = num_ref[...] / (den_ref[...] + 1e-6)
    h2 = h_ref[...] + jnp.maximum(hu_ref[...] + agg, 0.0)
    z = jnp.maximum(
        jnp.dot(h2, W1h_ref[...], preferred_element_type=_f32)
        + jnp.dot(act_ref[...], W1a_ref[...], preferred_element_type=_f32)
        + b1_ref[...], 0.0)
    y = jnp.dot(z, W2_ref[...], preferred_element_type=_f32) + b2_ref[...]

    @pl.when(i == 0)
    def _():
        out_ref[...] = jnp.zeros_like(out_ref)

    out_ref[...] += jnp.reshape(jnp.sum(y) / _N, (1, 1))


def _tc_head(h1, num1, den1, hu1, action, W1h, W1a, b1, W2, b2):
    n_blk = pl.BlockSpec((_BN, _H), lambda i: (i, 0))
    return pl.pallas_call(
        _head_body,
        grid=(_N // _BN,),
        in_specs=[
            n_blk, n_blk, n_blk, n_blk,
            pl.BlockSpec((_BN, _AD), lambda i: (i, 0)),
            pl.BlockSpec((_H, _H), lambda i: (0, 0)),
            pl.BlockSpec((_AD, _H), lambda i: (0, 0)),
            pl.BlockSpec((1, _H), lambda i: (0, 0)),
            pl.BlockSpec((_H, 1), lambda i: (0, 0)),
            pl.BlockSpec((1, 1), lambda i: (0, 0)),
        ],
        out_specs=pl.BlockSpec((1, 1), lambda i: (0, 0)),
        out_shape=jax.ShapeDtypeStruct((1, 1), _f32),
    )(h1, num1, den1, hu1, action, W1h, W1a, b1, W2, b2)


_MESH = plsc.VectorSubcoreMesh(core_axis_name="c", subcore_axis_name="s")


_GK = 5
_GCH = 40
_GSB = _GK * _GCH


def _sc_gather_body(td_hbm, ts_hbm, dst_hbm, src_hbm, gA_hbm, gBV_hbm,
                    idx_d, idx_s, bufA, bufBV, semi, semg, semw):
    c = lax.axis_index("c")
    s = lax.axis_index("s")
    wid = s * _NC + c
    ew = _E // _NW
    nsb = ew // _GSB

    def superchunk(i, carry):
        base = wid * ew + i * _GSB
        ci0 = pltpu.async_copy(dst_hbm.at[pl.ds(base, _GSB)], idx_d, semi)
        ci1 = pltpu.async_copy(src_hbm.at[pl.ds(base, _GSB)], idx_s, semi)
        ci0.wait()
        ci1.wait()
        descs = []
        for k in range(_GK):
            sl = pl.ds(k * _GCH, _GCH)
            descs.append(pltpu.async_copy(
                td_hbm.at[idx_d.at[sl]], bufA.at[sl], semg))
            descs.append(pltpu.async_copy(
                ts_hbm.at[idx_s.at[sl]], bufBV.at[sl], semg))
        for dsc in descs:
            dsc.wait()
        w0 = pltpu.async_copy(bufA, gA_hbm.at[pl.ds(base, _GSB)], semw)
        w1 = pltpu.async_copy(bufBV, gBV_hbm.at[pl.ds(base, _GSB)], semw)
        w0.wait()
        w1.wait()
        return carry

    lax.fori_loop(0, nsb, superchunk, 0)


@functools.partial(
    pl.kernel,
    out_type=[
        jax.ShapeDtypeStruct((_E, _H), _f32),
        jax.ShapeDtypeStruct((_E, 2 * _H), _f32),
    ],
    mesh=_MESH,
    scratch_types=[
        pltpu.VMEM((_GSB,), jnp.int32),
        pltpu.VMEM((_GSB,), jnp.int32),
        pltpu.VMEM((_GSB, _H), _f32),
        pltpu.VMEM((_GSB, 2 * _H), _f32),
        pltpu.SemaphoreType.DMA,
        pltpu.SemaphoreType.DMA,
        pltpu.SemaphoreType.DMA,
    ],
)
def _sc_gather(td_hbm, ts_hbm, dst_hbm, src_hbm, gA_hbm, gBV_hbm,
               idx_d, idx_s, bufA, bufBV, semi, semg, semw):
    _sc_gather_body(td_hbm, ts_hbm, dst_hbm, src_hbm, gA_hbm, gBV_hbm,
                    idx_d, idx_s, bufA, bufBV, semi, semg, semw)


_SK = 5
_SCH = 40
_SSB = _SK * _SCH


def _sc_scatter_stream(data_hbm, dst3_hbm, acc, dbuf, idxbuf, semd, sems, s):
    ew = _E // _NS
    nsb = ew // _SSB

    def superchunk(i, carry):
        base = s * ew + i * _SSB
        ld = pltpu.async_copy(data_hbm.at[pl.ds(base, _SSB)], dbuf, semd)
        li = pltpu.async_copy(dst3_hbm.at[pl.ds(base // _SCH, _SK)], idxbuf,
                              semd)
        ld.wait()
        li.wait()
        descs = []
        for k in range(_SK):
            descs.append(pltpu.async_copy(
                dbuf.at[pl.ds(k * _SCH, _SCH)], acc.at[idxbuf.at[k, 0]], sems,
                add=True))
        for dsc in descs:
            dsc.wait()
        return carry

    lax.fori_loop(0, nsb, superchunk, 0)


@functools.partial(
    pl.kernel,
    out_type=[
        jax.ShapeDtypeStruct((_NP, _H), _f32),
        jax.ShapeDtypeStruct((_NP, _H), _f32),
    ],
    mesh=_MESH,
    scratch_types=[
        pltpu.VMEM((_SSB, _H), _f32),
        pltpu.VMEM((_SK, 1, _SCH), jnp.int32),
        pltpu.VMEM_SHARED((_NP, _H), _f32),
        pltpu.SemaphoreType.DMA,
        pltpu.SemaphoreType.DMA,
    ],
)
def _sc_scatter(msg_hbm, sig_hbm, dst3_hbm, zeros_hbm, num_hbm, den_hbm,
                dbuf, idxbuf, acc, semd, sems):
    c = lax.axis_index("c")
    s = lax.axis_index("s")
    rows = _NP // _NS
    pltpu.sync_copy(zeros_hbm.at[pl.ds(s * rows, rows)],
                    acc.at[pl.ds(s * rows, rows)])
    plsc.subcore_barrier()

    @pl.when(c == 0)
    def _():
        _sc_scatter_stream(msg_hbm, dst3_hbm, acc, dbuf, idxbuf,
                           semd, sems, s)

    @pl.when(c == 1)
    def _():
        _sc_scatter_stream(sig_hbm, dst3_hbm, acc, dbuf, idxbuf,
                           semd, sems, s)

    plsc.subcore_barrier()

    @pl.when(c == 0)
    def _():
        pltpu.sync_copy(acc.at[pl.ds(s * rows, rows)],
                        num_hbm.at[pl.ds(s * rows, rows)])

    @pl.when(c == 1)
    def _():
        pltpu.sync_copy(acc.at[pl.ds(s * rows, rows)],
                        den_hbm.at[pl.ds(s * rows, rows)])


def kernel(h, e, edge_index, action, Wn, We, A, B, C, U, V, W1, b1, W2, b2):
    src = edge_index[0]
    dst = edge_index[1]
    dst3 = dst.reshape(_E // _SCH, 1, _SCH)
    zeros_n = jnp.zeros((_NP, _H), _f32)

    h0, td0, ts0, hu0 = _tc_proj(h, Wn, A[0], B[0], V[0], U[0])
    gA0, gBV0 = _sc_gather(td0, ts0, dst, src)
    msg0, sig0, eC1 = _tc_mid0(e, gA0, gBV0, We, C[0], C[1])
    num0, den0 = _sc_scatter(msg0, sig0, dst3, zeros_n)

    h1, td1, ts1, hu1 = _tc_upd_proj(h0, num0, den0, hu0,
                                     A[1], B[1], V[1], U[1])
    gA1, gBV1 = _sc_gather(td1, ts1, dst, src)
    msg1, sig1 = _tc_mid1(eC1, gA1, gBV1)
    num1, den1 = _sc_scatter(msg1, sig1, dst3, zeros_n)

    return _tc_head(h1, num1, den1, hu1, action,
                    W1[:_H], W1[_H:], b1.reshape(1, _H),
                    W2, b2.reshape(1, 1))
